# baseline (speedup 1.0000x reference)
import functools

import jax
import jax.numpy as jnp
from jax import lax
from jax.experimental import pallas as pl
from jax.experimental.pallas import tpu as pltpu
from jax.experimental.pallas import tpu_sc as plsc

_NC = 2
_NS = 16
_NW = _NC * _NS
_L = 16
_D = 32
_B = 16384
_BPW = _B // _NW
_CH = 128
_NCH = _BPW // _CH


def _mf_body(p1_hbm, p2_hbm, P_hbm, Q_hbm, b1_hbm, b2_hbm, out_hbm,
             idx1_v, idx2_v, p_v, q_v, b1_v, b2_v, out_v, sem):
    wid = lax.axis_index("s") * _NC + lax.axis_index("c")

    pltpu.sync_copy(p1_hbm.at[pl.ds(wid * _NCH, _NCH)], idx1_v)
    pltpu.sync_copy(p2_hbm.at[pl.ds(wid * _NCH, _NCH)], idx2_v)

    copies = []
    for j in range(_NCH):
        sl = pl.ds(j * _CH, _CH)
        copies.append(pltpu.async_copy(P_hbm.at[idx1_v.at[j]], p_v.at[sl], sem))
        copies.append(pltpu.async_copy(Q_hbm.at[idx2_v.at[j]], q_v.at[sl], sem))
        copies.append(pltpu.async_copy(b1_hbm.at[idx1_v.at[j]], b1_v.at[sl], sem))
        copies.append(pltpu.async_copy(b2_hbm.at[idx2_v.at[j]], b2_v.at[sl], sem))
    for c in copies:
        c.wait()

    lane = lax.iota(jnp.int32, _L)
    cols = [jnp.full((_L,), j, dtype=jnp.int32) for j in range(_D)]

    def group(g, carry):
        rows = g * _L + lane
        acc = b1_v[pl.ds(g * _L, _L)] + b2_v[pl.ds(g * _L, _L)]
        for j in range(_D):
            acc = acc + plsc.load_gather(p_v, [rows, cols[j]]) * \
                plsc.load_gather(q_v, [rows, cols[j]])
        out_v[pl.ds(g * _L, _L)] = acc
        return carry

    lax.fori_loop(0, _BPW // _L, group, 0)

    pltpu.sync_copy(out_v, out_hbm.at[pl.ds(wid * _BPW, _BPW)])


@jax.jit
def kernel(player1, player2, P, Q, player1_bias, player2_bias):
    p1 = player1.astype(jnp.int32).reshape(_NW * _NCH, _CH)
    p2 = player2.astype(jnp.int32).reshape(_NW * _NCH, _CH)
    b1 = player1_bias.reshape(-1)
    b2 = player2_bias.reshape(-1)
    mesh = plsc.VectorSubcoreMesh(core_axis_name="c", subcore_axis_name="s")
    f = pl.kernel(
        _mf_body,
        out_type=jax.ShapeDtypeStruct((_B,), jnp.float32),
        mesh=mesh,
        compiler_params=pltpu.CompilerParams(
            needs_layout_passes=False, use_tc_tiling_on_sc=False),
        scratch_types=[
            pltpu.VMEM((_NCH, _CH), jnp.int32),
            pltpu.VMEM((_NCH, _CH), jnp.int32),
            pltpu.VMEM((_BPW, _D), jnp.float32),
            pltpu.VMEM((_BPW, _D), jnp.float32),
            pltpu.VMEM((_BPW,), jnp.float32),
            pltpu.VMEM((_BPW,), jnp.float32),
            pltpu.VMEM((_BPW,), jnp.float32),
            pltpu.SemaphoreType.DMA,
        ],
    )
    return f(p1, p2, P, Q, b1, b2)
